# Initial kernel scaffold; baseline (speedup 1.0000x reference)
import jax
import jax.numpy as jnp
from jax.experimental import pallas as pl


def kernel(nodes, edges, W, b, gamma, beta):
    raise NotImplementedError("write your pallas kernel here")


# speedup vs baseline: 22.5150x; 22.5150x over previous
import functools

import jax
import jax.numpy as jnp
from jax import lax
from jax.experimental import pallas as pl
from jax.experimental.pallas import tpu as pltpu
from jax.experimental.pallas import tpu_sc as plsc

N = 10000
E = 320000
D = 128

NC = 2
NS = 16
CHUNK = 128
NCHUNK = E // CHUNK
BASE_CHUNKS = NCHUNK // (NC * NS)
EXTRA = NCHUNK - BASE_CHUNKS * NC * NS

DEG_PAD = 10240
RBLK = 128
NRBLK = N // RBLK

_SC_MESH = plsc.VectorSubcoreMesh(
    core_axis_name="c", subcore_axis_name="s", num_cores=NC, num_subcores=NS)

_f32 = jnp.float32


def _zero_vec_ref(ref, n):
    zero = jnp.zeros((16,), _f32)

    def body(i, _):
        ref[pl.ds(i * 16, 16)] = zero
        return 0

    lax.fori_loop(0, n // 16, body, 0, unroll=4)


@functools.partial(
    pl.kernel,
    out_type=jax.ShapeDtypeStruct((NC * DEG_PAD,), _f32),
    mesh=_SC_MESH,
    scratch_types=[
        pltpu.VMEM((CHUNK,), jnp.int32),
        pltpu.VMEM((CHUNK,), _f32),
        pltpu.VMEM((DEG_PAD // NS,), _f32),
        pltpu.VMEM_SHARED((DEG_PAD,), _f32),
    ],
)
def _sc_deg(dst_hbm, deg_out, dstv, onesv, zv, deg_sh):
    c = lax.axis_index("c")
    s = lax.axis_index("s")
    wid = s * NC + c

    one = jnp.full((16,), 1.0, _f32)
    for i in range(CHUNK // 16):
        onesv[pl.ds(i * 16, 16)] = one
    _zero_vec_ref(zv, DEG_PAD // NS)

    pltpu.sync_copy(zv, deg_sh.at[pl.ds(s * (DEG_PAD // NS), DEG_PAD // NS)])
    plsc.subcore_barrier()

    nchunk = BASE_CHUNKS + jnp.where(wid < EXTRA, 1, 0)

    def body(i, _):
        g = wid + i * (NC * NS)
        pltpu.sync_copy(dst_hbm.at[g], dstv)
        pltpu.sync_copy(onesv, deg_sh.at[dstv], add=True)
        return 0

    lax.fori_loop(0, nchunk, body, 0)
    plsc.subcore_barrier()

    pltpu.sync_copy(
        deg_sh.at[pl.ds(s * (DEG_PAD // NS), DEG_PAD // NS)],
        deg_out.at[pl.ds(c * DEG_PAD + s * (DEG_PAD // NS), DEG_PAD // NS)])


@functools.partial(
    pl.kernel,
    out_type=jax.ShapeDtypeStruct((NC * N, D), _f32),
    mesh=_SC_MESH,
    scratch_types=[
        pltpu.VMEM((CHUNK,), jnp.int32),
        pltpu.VMEM((CHUNK,), jnp.int32),
        pltpu.VMEM((CHUNK, D), _f32),
        pltpu.VMEM_SHARED((N, D), _f32),
        pltpu.SemaphoreType.DMA,
    ],
)
def _sc_agg(src_hbm, dst_hbm, y_hbm, acc_out, srcv, dstv, rows, acc_sh, sem):
    c = lax.axis_index("c")
    s = lax.axis_index("s")
    wid = s * NC + c

    zero = jnp.zeros((16,), _f32)

    def zbody(i, _):
        r = i >> 3
        col = (i & 7) * 16
        rows[r, pl.ds(col, 16)] = zero
        return 0

    lax.fori_loop(0, CHUNK * (D // 16), zbody, 0, unroll=8)

    nrb = 4 + jnp.where(s < NRBLK - 4 * NS, 1, 0)

    def zcopy(i, _):
        pltpu.sync_copy(rows, acc_sh.at[pl.ds((s + i * NS) * RBLK, RBLK)])
        return 0

    lax.fori_loop(0, nrb, zcopy, 0)

    @pl.when(s == NS - 1)
    def _():
        pltpu.sync_copy(rows.at[pl.ds(0, N - NRBLK * RBLK)],
                        acc_sh.at[pl.ds(NRBLK * RBLK, N - NRBLK * RBLK)])

    plsc.subcore_barrier()

    nchunk = BASE_CHUNKS + jnp.where(wid < EXTRA, 1, 0)

    def body(i, _):
        g = wid + i * (NC * NS)
        pltpu.sync_copy(src_hbm.at[g], srcv)
        pltpu.sync_copy(dst_hbm.at[g], dstv)
        pltpu.async_copy(y_hbm.at[srcv], rows, sem).wait()
        pltpu.sync_copy(rows, acc_sh.at[dstv], add=True)
        return 0

    lax.fori_loop(0, nchunk, body, 0)
    plsc.subcore_barrier()

    def wcopy(i, _):
        r0 = (s + i * NS) * RBLK
        pltpu.sync_copy(acc_sh.at[pl.ds(r0, RBLK)],
                        acc_out.at[pl.ds(c * N + r0, RBLK)])
        return 0

    lax.fori_loop(0, nrb, wcopy, 0)

    @pl.when(s == NS - 1)
    def _():
        tail = N - NRBLK * RBLK
        pltpu.sync_copy(acc_sh.at[pl.ds(NRBLK * RBLK, tail)],
                        acc_out.at[pl.ds(c * N + NRBLK * RBLK, tail)])


_BLK = 2000
_GRID = N // _BLK


def _prep_body(nodes_ref, w_ref, ds_ref, y_ref):
    dinv = lax.rsqrt(ds_ref[...] + 1.0)
    x = lax.dot_general(nodes_ref[...], w_ref[...],
                        (((1,), (1,)), ((), ())),
                        preferred_element_type=_f32)
    y_ref[...] = x * dinv


_tc_prep = pl.pallas_call(
    _prep_body,
    grid=(_GRID,),
    in_specs=[
        pl.BlockSpec((_BLK, D), lambda i: (i, 0)),
        pl.BlockSpec((D, D), lambda i: (0, 0)),
        pl.BlockSpec((_BLK, 1), lambda i: (i, 0)),
    ],
    out_specs=pl.BlockSpec((_BLK, D), lambda i: (i, 0)),
    out_shape=jax.ShapeDtypeStruct((N, D), _f32),
)


def _post_body(a0_ref, a1_ref, y_ref, ds_ref, b_ref, g_ref, be_ref, o_ref):
    dinv = lax.rsqrt(ds_ref[...] + 1.0)
    pre = (a0_ref[...] + a1_ref[...] + y_ref[...]) * dinv + b_ref[...]
    mu = jnp.mean(pre, axis=-1, keepdims=True)
    dev = pre - mu
    var = jnp.mean(dev * dev, axis=-1, keepdims=True)
    o = dev * lax.rsqrt(var + 1e-5) * g_ref[...] + be_ref[...]
    o_ref[...] = jnp.maximum(o, 0.0)


_tc_post = pl.pallas_call(
    _post_body,
    grid=(_GRID,),
    in_specs=[
        pl.BlockSpec((_BLK, D), lambda i: (i, 0)),
        pl.BlockSpec((_BLK, D), lambda i: (i + _GRID, 0)),
        pl.BlockSpec((_BLK, D), lambda i: (i, 0)),
        pl.BlockSpec((_BLK, 1), lambda i: (i, 0)),
        pl.BlockSpec((1, D), lambda i: (0, 0)),
        pl.BlockSpec((1, D), lambda i: (0, 0)),
        pl.BlockSpec((1, D), lambda i: (0, 0)),
    ],
    out_specs=pl.BlockSpec((_BLK, D), lambda i: (i, 0)),
    out_shape=jax.ShapeDtypeStruct((N, D), _f32),
)


def kernel(nodes, edges, W, b, gamma, beta):
    e = edges.astype(jnp.int32)
    src2 = e[0].reshape(NCHUNK, CHUNK)
    dst2 = e[1].reshape(NCHUNK, CHUNK)

    degp = _sc_deg(dst2)
    dsum = (degp[:N] + degp[DEG_PAD:DEG_PAD + N]).reshape(N, 1)

    y = _tc_prep(nodes, W, dsum)
    accp = _sc_agg(src2, dst2, y)
    out = _tc_post(accp, accp, y, dsum,
                   b.reshape(1, D), gamma.reshape(1, D), beta.reshape(1, D))
    return out


# speedup vs baseline: 46.1892x; 2.0515x over previous
import functools

import jax
import jax.numpy as jnp
from jax import lax
from jax.experimental import pallas as pl
from jax.experimental.pallas import tpu as pltpu
from jax.experimental.pallas import tpu_sc as plsc

N = 10000
E = 320000
D = 128

NC = 2
NS = 16
NW = NC * NS
CHUNK = 128
CPT = 80
EP = NW * CPT * CHUNK
NBUF = 2
GU = 4
NGRP = CPT // GU

N_PAD = N + 16
DEG_PAD = 10240
RBLK = 128
NRBLK = N // RBLK

_SC_MESH = plsc.VectorSubcoreMesh(
    core_axis_name="c", subcore_axis_name="s", num_cores=NC, num_subcores=NS)

_f32 = jnp.float32


@functools.partial(
    pl.kernel,
    out_type=jax.ShapeDtypeStruct((NC * DEG_PAD,), _f32),
    mesh=_SC_MESH,
    scratch_types=[
        pltpu.VMEM((CPT, CHUNK), jnp.int32),
        pltpu.VMEM((CHUNK,), _f32),
        pltpu.VMEM((DEG_PAD // NS,), _f32),
        pltpu.VMEM_SHARED((DEG_PAD,), _f32),
        pltpu.SemaphoreType.DMA,
    ],
)
def _sc_deg(dst_hbm, deg_out, dstall, onesv, zv, deg_sh, ssem):
    c = lax.axis_index("c")
    s = lax.axis_index("s")
    wid = s * NC + c

    one = jnp.full((16,), 1.0, _f32)
    for i in range(CHUNK // 16):
        onesv[pl.ds(i * 16, 16)] = one
    zero = jnp.zeros((16,), _f32)

    def zfill(i, _):
        zv[pl.ds(i * 16, 16)] = zero
        return 0

    lax.fori_loop(0, DEG_PAD // NS // 16, zfill, 0, unroll=4)

    pltpu.sync_copy(zv, deg_sh.at[pl.ds(s * (DEG_PAD // NS), DEG_PAD // NS)])
    pltpu.sync_copy(dst_hbm.at[wid], dstall)
    plsc.subcore_barrier()

    def fire(i, _):
        pltpu.async_copy(onesv, deg_sh.at[dstall.at[i]], ssem, add=True)
        return 0

    lax.fori_loop(0, CPT, fire, 0)

    def drain(i, _):
        pltpu.make_async_copy(dst_hbm.at[0, 0], dstall.at[0], ssem).wait()
        return 0

    lax.fori_loop(0, CPT, drain, 0)
    plsc.subcore_barrier()

    pltpu.sync_copy(
        deg_sh.at[pl.ds(s * (DEG_PAD // NS), DEG_PAD // NS)],
        deg_out.at[pl.ds(c * DEG_PAD + s * (DEG_PAD // NS), DEG_PAD // NS)])


NIDX = 4


@functools.partial(
    pl.kernel,
    out_type=jax.ShapeDtypeStruct((NC * N, D), _f32),
    mesh=_SC_MESH,
    scratch_types=[
        [pltpu.VMEM((CHUNK, D), _f32)] * NBUF,
        [pltpu.VMEM((CHUNK,), jnp.int32)] * NIDX,
        [pltpu.VMEM((CHUNK,), jnp.int32)] * NIDX,
        [pltpu.SemaphoreType.DMA] * NBUF,
        [pltpu.SemaphoreType.DMA] * NIDX,
        pltpu.VMEM_SHARED((N_PAD, D), _f32),
    ],
)
def _sc_agg(src_hbm, dst_hbm, y_hbm, acc_out,
            rows, srcb, dstb, gsem, isem, acc_sh):
    c = lax.axis_index("c")
    s = lax.axis_index("s")
    wid = s * NC + c

    zero = jnp.zeros((16,), _f32)

    def zbody(i, _):
        rows[0][i >> 3, pl.ds((i & 7) * 16, 16)] = zero
        return 0

    lax.fori_loop(0, CHUNK * (D // 16), zbody, 0, unroll=8)

    nrb = 4 + jnp.where(s < NRBLK - 4 * NS, 1, 0)

    def zcopy(i, _):
        pltpu.sync_copy(rows[0], acc_sh.at[pl.ds((s + i * NS) * RBLK, RBLK)])
        return 0

    lax.fori_loop(0, nrb, zcopy, 0)

    @pl.when(s == NS - 1)
    def _():
        pltpu.sync_copy(rows[0].at[pl.ds(0, N - NRBLK * RBLK)],
                        acc_sh.at[pl.ds(NRBLK * RBLK, N - NRBLK * RBLK)])

    plsc.subcore_barrier()

    def _fire_idx(cc, k):
        pltpu.async_copy(src_hbm.at[wid, cc], srcb[k], isem[k])
        pltpu.async_copy(dst_hbm.at[wid, cc], dstb[k], isem[k])

    def _drain_idx(k):
        pltpu.make_async_copy(src_hbm.at[0, 0], srcb[k], isem[k]).wait()
        pltpu.make_async_copy(src_hbm.at[0, 0], dstb[k], isem[k]).wait()

    def _drain_gather(b):
        pltpu.make_async_copy(y_hbm.at[pl.ds(0, CHUNK)], rows[b],
                              gsem[b]).wait()

    _fire_idx(0, 0)
    _fire_idx(1, 1)
    _drain_idx(0)
    pltpu.async_copy(y_hbm.at[srcb[0]], rows[0], gsem[0])

    def group(g, _):
        for j in range(GU):
            cc = g * GU + j

            @pl.when(cc + 2 < CPT)
            def _(cc=cc, j=j):
                _fire_idx(cc + 2, (j + 2) % NIDX)

            @pl.when(cc + 1 < CPT)
            def _(cc=cc, j=j):
                _drain_idx((j + 1) % NIDX)
                pltpu.async_copy(y_hbm.at[srcb[(j + 1) % NIDX]],
                                 rows[(j + 1) % NBUF], gsem[(j + 1) % NBUF])

            _drain_gather(j % NBUF)
            pltpu.sync_copy(rows[j % NBUF], acc_sh.at[dstb[j % NIDX]],
                            add=True)
        return 0

    lax.fori_loop(0, NGRP, group, 0)
    plsc.subcore_barrier()

    def wcopy(i, _):
        r0 = (s + i * NS) * RBLK
        pltpu.sync_copy(acc_sh.at[pl.ds(r0, RBLK)],
                        acc_out.at[pl.ds(c * N + r0, RBLK)])
        return 0

    lax.fori_loop(0, nrb, wcopy, 0)

    @pl.when(s == NS - 1)
    def _():
        tail = N - NRBLK * RBLK
        pltpu.sync_copy(acc_sh.at[pl.ds(NRBLK * RBLK, tail)],
                        acc_out.at[pl.ds(c * N + NRBLK * RBLK, tail)])


_BLK = 2000
_GRID = N // _BLK


def _prep_body(nodes_ref, w_ref, ds_ref, y_ref):
    dinv = lax.rsqrt(ds_ref[...] + 1.0)
    x = lax.dot_general(nodes_ref[...], w_ref[...],
                        (((1,), (1,)), ((), ())),
                        preferred_element_type=_f32)
    y_ref[...] = x * dinv


_tc_prep = pl.pallas_call(
    _prep_body,
    grid=(_GRID,),
    in_specs=[
        pl.BlockSpec((_BLK, D), lambda i: (i, 0)),
        pl.BlockSpec((D, D), lambda i: (0, 0)),
        pl.BlockSpec((_BLK, 1), lambda i: (i, 0)),
    ],
    out_specs=pl.BlockSpec((_BLK, D), lambda i: (i, 0)),
    out_shape=jax.ShapeDtypeStruct((N, D), _f32),
)


def _post_body(a0_ref, a1_ref, y_ref, ds_ref, b_ref, g_ref, be_ref, o_ref):
    dinv = lax.rsqrt(ds_ref[...] + 1.0)
    pre = (a0_ref[...] + a1_ref[...] + y_ref[...]) * dinv + b_ref[...]
    mu = jnp.mean(pre, axis=-1, keepdims=True)
    dev = pre - mu
    var = jnp.mean(dev * dev, axis=-1, keepdims=True)
    o = dev * lax.rsqrt(var + 1e-5) * g_ref[...] + be_ref[...]
    o_ref[...] = jnp.maximum(o, 0.0)


_tc_post = pl.pallas_call(
    _post_body,
    grid=(_GRID,),
    in_specs=[
        pl.BlockSpec((_BLK, D), lambda i: (i, 0)),
        pl.BlockSpec((_BLK, D), lambda i: (i + _GRID, 0)),
        pl.BlockSpec((_BLK, D), lambda i: (i, 0)),
        pl.BlockSpec((_BLK, 1), lambda i: (i, 0)),
        pl.BlockSpec((1, D), lambda i: (0, 0)),
        pl.BlockSpec((1, D), lambda i: (0, 0)),
        pl.BlockSpec((1, D), lambda i: (0, 0)),
    ],
    out_specs=pl.BlockSpec((_BLK, D), lambda i: (i, 0)),
    out_shape=jax.ShapeDtypeStruct((N, D), _f32),
)


def kernel(nodes, edges, W, b, gamma, beta):
    e = edges.astype(jnp.int32)
    npad = EP - E
    pad_src = jnp.arange(npad, dtype=jnp.int32) % CHUNK
    pad_dst = jnp.arange(npad, dtype=jnp.int32) % 16 + N
    src3 = jnp.concatenate([e[0], pad_src]).reshape(NW, CPT, CHUNK)
    dst3 = jnp.concatenate([e[1], pad_dst]).reshape(NW, CPT, CHUNK)

    degp = _sc_deg(dst3)
    dsum = (degp[:N] + degp[DEG_PAD:DEG_PAD + N]).reshape(N, 1)

    y = _tc_prep(nodes, W, dsum)
    accp = _sc_agg(src3, dst3, y)
    out = _tc_post(accp, accp, y, dsum,
                   b.reshape(1, D), gamma.reshape(1, D), beta.reshape(1, D))
    return out


# speedup vs baseline: 48.7851x; 1.0562x over previous
import functools

import jax
import jax.numpy as jnp
from jax import lax
from jax.experimental import pallas as pl
from jax.experimental.pallas import tpu as pltpu
from jax.experimental.pallas import tpu_sc as plsc

N = 10000
E = 320000
D = 128

NC = 2
NS = 16
NW = NC * NS
CHUNK = 128
NCH = E // CHUNK
GU = 12
LOOP = 84
NROW = 3
NIDX = 4

DEG_PAD = 10240
RBLK = 128
NRBLK = N // RBLK

_SC_MESH = plsc.VectorSubcoreMesh(
    core_axis_name="c", subcore_axis_name="s", num_cores=NC, num_subcores=NS)

_f32 = jnp.float32


@functools.partial(
    pl.kernel,
    out_type=jax.ShapeDtypeStruct((NC * DEG_PAD,), _f32),
    mesh=_SC_MESH,
    scratch_types=[
        [pltpu.VMEM((CHUNK,), jnp.int32)] * NIDX,
        pltpu.VMEM((CHUNK,), _f32),
        pltpu.VMEM((DEG_PAD // NS,), _f32),
        pltpu.VMEM_SHARED((DEG_PAD,), _f32),
        [pltpu.SemaphoreType.DMA] * NIDX,
        [pltpu.SemaphoreType.DMA] * NIDX,
    ],
)
def _sc_deg(edg_hbm, deg_out, dstb, onesv, zv, deg_sh, isem, ssem):
    c = lax.axis_index("c")
    s = lax.axis_index("s")
    wid = s * NC + c

    one = jnp.full((16,), 1.0, _f32)
    for i in range(CHUNK // 16):
        onesv[pl.ds(i * 16, 16)] = one
    zero = jnp.zeros((16,), _f32)

    def zfill(i, _):
        zv[pl.ds(i * 16, 16)] = zero
        return 0

    lax.fori_loop(0, DEG_PAD // NS // 16, zfill, 0, unroll=4)

    def fire_idx(cc, k):
        pltpu.async_copy(edg_hbm.at[wid + cc * NW, 1], dstb[k], isem[k])

    def drain_idx(k):
        pltpu.make_async_copy(edg_hbm.at[0, 0], dstb[k], isem[k]).wait()

    def drain_scat(k):
        pltpu.make_async_copy(edg_hbm.at[0, 0], dstb[k], ssem[k]).wait()

    fire_idx(0, 0)
    fire_idx(1, 1)
    pltpu.sync_copy(zv, deg_sh.at[pl.ds(s * (DEG_PAD // NS), DEG_PAD // NS)])
    plsc.subcore_barrier()

    def body(g, _):
        for j in range(NIDX):
            cc = g * NIDX + j
            alive0 = wid + cc * NW < NCH
            alive2 = wid + (cc + 2) * NW < NCH

            @pl.when((cc >= 2) & alive0)
            def _(j=j):
                drain_scat((j - 2) % NIDX)

            @pl.when(alive2)
            def _(cc=cc, j=j):
                fire_idx(cc + 2, (j + 2) % NIDX)

            @pl.when(alive0)
            def _(j=j):
                drain_idx(j)
                pltpu.async_copy(onesv, deg_sh.at[dstb[j]], ssem[j],
                                 add=True)
        return 0

    lax.fori_loop(0, LOOP // NIDX, body, 0)

    tmax = (NCH - 1 - wid) // NW
    for j in range(NIDX):
        @pl.when((tmax % NIDX == j) | ((tmax - 1) % NIDX == j))
        def _(j=j):
            drain_scat(j)

    plsc.subcore_barrier()
    pltpu.sync_copy(
        deg_sh.at[pl.ds(s * (DEG_PAD // NS), DEG_PAD // NS)],
        deg_out.at[pl.ds(c * DEG_PAD + s * (DEG_PAD // NS), DEG_PAD // NS)])


@functools.partial(
    pl.kernel,
    out_type=jax.ShapeDtypeStruct((NC * N, D), _f32),
    mesh=_SC_MESH,
    scratch_types=[
        [pltpu.VMEM((CHUNK, D), _f32)] * NROW,
        [pltpu.VMEM((2, CHUNK), jnp.int32)] * NIDX,
        [pltpu.SemaphoreType.DMA] * NROW,
        [pltpu.SemaphoreType.DMA] * NIDX,
        [pltpu.SemaphoreType.DMA] * NROW,
        pltpu.VMEM_SHARED((N, D), _f32),
    ],
)
def _sc_agg(edg_hbm, y_hbm, acc_out, rows, idxb, gsem, isem, ssem, acc_sh):
    c = lax.axis_index("c")
    s = lax.axis_index("s")
    wid = s * NC + c

    def fire_idx(cc, k):
        pltpu.async_copy(edg_hbm.at[wid + cc * NW], idxb[k], isem[k])

    def drain_idx(k):
        pltpu.make_async_copy(edg_hbm.at[0], idxb[k], isem[k]).wait()

    def fire_gather(k, b):
        pltpu.async_copy(y_hbm.at[idxb[k].at[0]], rows[b], gsem[b])

    def drain_gather(b):
        pltpu.make_async_copy(y_hbm.at[pl.ds(0, CHUNK)], rows[b],
                              gsem[b]).wait()

    def fire_scat(k, b):
        pltpu.async_copy(rows[b], acc_sh.at[idxb[k].at[1]], ssem[b],
                         add=True)

    def drain_scat(b):
        pltpu.make_async_copy(y_hbm.at[pl.ds(0, CHUNK)], rows[b],
                              ssem[b]).wait()

    fire_idx(0, 0)
    fire_idx(1, 1)

    zero = jnp.zeros((16,), _f32)
    zb = rows[NROW - 1]

    def zbody(i, _):
        zb[i >> 3, pl.ds((i & 7) * 16, 16)] = zero
        return 0

    lax.fori_loop(0, CHUNK * (D // 16), zbody, 0, unroll=8)

    nrb = 4 + jnp.where(s < NRBLK - 4 * NS, 1, 0)

    def zcopy(i, _):
        pltpu.sync_copy(zb, acc_sh.at[pl.ds((s + i * NS) * RBLK, RBLK)])
        return 0

    lax.fori_loop(0, nrb, zcopy, 0)

    @pl.when(s == NS - 1)
    def _():
        pltpu.sync_copy(zb.at[pl.ds(0, N - NRBLK * RBLK)],
                        acc_sh.at[pl.ds(NRBLK * RBLK, N - NRBLK * RBLK)])

    plsc.subcore_barrier()

    drain_idx(0)
    fire_gather(0, 0)

    def body(g, _):
        for j in range(GU):
            cc = g * GU + j
            alive0 = wid + cc * NW < NCH
            alive1 = wid + (cc + 1) * NW < NCH
            alive2 = wid + (cc + 2) * NW < NCH

            @pl.when((cc >= 2) & alive0)
            def _(j=j):
                drain_scat((j - 2) % NROW)

            @pl.when(alive2)
            def _(cc=cc, j=j):
                fire_idx(cc + 2, (j + 2) % NIDX)

            @pl.when(alive1)
            def _(j=j):
                drain_idx((j + 1) % NIDX)
                fire_gather((j + 1) % NIDX, (j + 1) % NROW)

            @pl.when(alive0)
            def _(j=j):
                drain_gather(j % NROW)
                fire_scat(j % NIDX, j % NROW)
        return 0

    lax.fori_loop(0, LOOP // GU, body, 0)

    tmax = (NCH - 1 - wid) // NW
    for b in range(NROW):
        @pl.when((tmax % NROW == b) | ((tmax - 1) % NROW == b))
        def _(b=b):
            drain_scat(b)

    plsc.subcore_barrier()

    def wcopy(i, _):
        r0 = (s + i * NS) * RBLK
        pltpu.sync_copy(acc_sh.at[pl.ds(r0, RBLK)],
                        acc_out.at[pl.ds(c * N + r0, RBLK)])
        return 0

    lax.fori_loop(0, nrb, wcopy, 0)

    @pl.when(s == NS - 1)
    def _():
        tail = N - NRBLK * RBLK
        pltpu.sync_copy(acc_sh.at[pl.ds(NRBLK * RBLK, tail)],
                        acc_out.at[pl.ds(c * N + NRBLK * RBLK, tail)])


_BLK = 2000
_GRID = N // _BLK


def _prep_body(nodes_ref, w_ref, ds_ref, y_ref):
    dinv = lax.rsqrt(ds_ref[...] + 1.0)
    x = lax.dot_general(nodes_ref[...], w_ref[...],
                        (((1,), (1,)), ((), ())),
                        preferred_element_type=_f32)
    y_ref[...] = x * dinv


_tc_prep = pl.pallas_call(
    _prep_body,
    grid=(_GRID,),
    in_specs=[
        pl.BlockSpec((_BLK, D), lambda i: (i, 0)),
        pl.BlockSpec((D, D), lambda i: (0, 0)),
        pl.BlockSpec((_BLK, 1), lambda i: (i, 0)),
    ],
    out_specs=pl.BlockSpec((_BLK, D), lambda i: (i, 0)),
    out_shape=jax.ShapeDtypeStruct((N, D), _f32),
)


def _post_body(a0_ref, a1_ref, y_ref, ds_ref, b_ref, g_ref, be_ref, o_ref):
    dinv = lax.rsqrt(ds_ref[...] + 1.0)
    pre = (a0_ref[...] + a1_ref[...] + y_ref[...]) * dinv + b_ref[...]
    mu = jnp.mean(pre, axis=-1, keepdims=True)
    dev = pre - mu
    var = jnp.mean(dev * dev, axis=-1, keepdims=True)
    o = dev * lax.rsqrt(var + 1e-5) * g_ref[...] + be_ref[...]
    o_ref[...] = jnp.maximum(o, 0.0)


_tc_post = pl.pallas_call(
    _post_body,
    grid=(_GRID,),
    in_specs=[
        pl.BlockSpec((_BLK, D), lambda i: (i, 0)),
        pl.BlockSpec((_BLK, D), lambda i: (i + _GRID, 0)),
        pl.BlockSpec((_BLK, D), lambda i: (i, 0)),
        pl.BlockSpec((_BLK, 1), lambda i: (i, 0)),
        pl.BlockSpec((1, D), lambda i: (0, 0)),
        pl.BlockSpec((1, D), lambda i: (0, 0)),
        pl.BlockSpec((1, D), lambda i: (0, 0)),
    ],
    out_specs=pl.BlockSpec((_BLK, D), lambda i: (i, 0)),
    out_shape=jax.ShapeDtypeStruct((N, D), _f32),
)


def kernel(nodes, edges, W, b, gamma, beta):
    e = edges.astype(jnp.int32)
    edg = jnp.stack([e[0].reshape(NCH, CHUNK), e[1].reshape(NCH, CHUNK)],
                    axis=1)

    degp = _sc_deg(edg)
    dsum = (degp[:N] + degp[DEG_PAD:DEG_PAD + N]).reshape(N, 1)

    y = _tc_prep(nodes, W, dsum)
    accp = _sc_agg(edg, y)
    out = _tc_post(accp, accp, y, dsum,
                   b.reshape(1, D), gamma.reshape(1, D), beta.reshape(1, D))
    return out
